# baseline (speedup 1.0000x reference)
import functools

import jax
import jax.numpy as jnp
from jax.experimental import pallas as pl
from jax.experimental.pallas import tpu as pltpu

_BR = 128


def _body(dx_ref, inp_ref, proc_ref, demo_ref,
          wdx_ref, bdx_ref, winp_ref, binp_ref, wproc_ref, bproc_ref,
          wdemo_ref, bdemo_ref, wf_i_ref, wf_p_ref, wf_d_ref, bf_ref,
          dx0_ref, out_ref):
    R = dx_ref.shape[0]
    T = inp_ref.shape[1]

    dx0 = jnp.tanh(
        jnp.dot(dx_ref[...], wdx_ref[...], preferred_element_type=jnp.float32)
        + bdx_ref[...])
    dx0_ref[...] = dx0

    iv = inp_ref.shape[2]
    pv = proc_ref.shape[2]
    ie = jnp.tanh(
        jnp.dot(inp_ref[...].reshape(R * T, iv), winp_ref[...],
                preferred_element_type=jnp.float32) + binp_ref[...])
    pe = jnp.tanh(
        jnp.dot(proc_ref[...].reshape(R * T, pv), wproc_ref[...],
                preferred_element_type=jnp.float32) + bproc_ref[...])
    de = jnp.tanh(
        jnp.dot(demo_ref[...], wdemo_ref[...],
                preferred_element_type=jnp.float32) + bdemo_ref[...])

    fused = (jnp.dot(ie, wf_i_ref[...], preferred_element_type=jnp.float32)
             + jnp.dot(pe, wf_p_ref[...], preferred_element_type=jnp.float32))
    fused = fused.reshape(R, T, -1)
    demo_part = (jnp.dot(de, wf_d_ref[...], preferred_element_type=jnp.float32)
                 + bf_ref[...])
    out_ref[...] = jnp.tanh(fused + demo_part[:, None, :])


def kernel(dx_history, inp, proc, demo, W_dx, b_dx, W_inp, b_inp,
           W_proc, b_proc, W_demo, b_demo, W_f, b_f):
    B, DXV = dx_history.shape
    _, T, IV = inp.shape
    PV = proc.shape[2]
    DD = demo.shape[1]
    DXE = W_dx.shape[0]
    IE = W_inp.shape[0]
    PE = W_proc.shape[0]
    DE = W_demo.shape[0]
    FE = W_f.shape[0]

    wdx_t = W_dx.T
    winp_t = W_inp.T
    wproc_t = W_proc.T
    wdemo_t = W_demo.T
    wf_i = W_f[:, :IE].T
    wf_p = W_f[:, IE:IE + PE].T
    wf_d = W_f[:, IE + PE:].T
    bdx2 = b_dx.reshape(1, DXE)
    binp2 = b_inp.reshape(1, IE)
    bproc2 = b_proc.reshape(1, PE)
    bdemo2 = b_demo.reshape(1, DE)
    bf2 = b_f.reshape(1, FE)

    grid = (B // _BR,)
    full = lambda shape: pl.BlockSpec(shape, lambda i: (0,) * len(shape))

    dx0, out = pl.pallas_call(
        _body,
        grid=grid,
        in_specs=[
            pl.BlockSpec((_BR, DXV), lambda i: (i, 0)),
            pl.BlockSpec((_BR, T, IV), lambda i: (i, 0, 0)),
            pl.BlockSpec((_BR, T, PV), lambda i: (i, 0, 0)),
            pl.BlockSpec((_BR, DD), lambda i: (i, 0)),
            full((DXV, DXE)), full((1, DXE)),
            full((IV, IE)), full((1, IE)),
            full((PV, PE)), full((1, PE)),
            full((DD, DE)), full((1, DE)),
            full((IE, FE)), full((PE, FE)), full((DE, FE)), full((1, FE)),
        ],
        out_specs=[
            pl.BlockSpec((_BR, DXE), lambda i: (i, 0)),
            pl.BlockSpec((_BR, T, FE), lambda i: (i, 0, 0)),
        ],
        out_shape=[
            jax.ShapeDtypeStruct((B, DXE), jnp.float32),
            jax.ShapeDtypeStruct((B, T, FE), jnp.float32),
        ],
        compiler_params=pltpu.CompilerParams(
            dimension_semantics=("arbitrary",),
        ),
    )(dx_history, inp, proc, demo,
      wdx_t, bdx2, winp_t, binp2, wproc_t, bproc2,
      wdemo_t, bdemo2, wf_i, wf_p, wf_d, bf2)
    return dx0, out


# speedup vs baseline: 4.0351x; 4.0351x over previous
import jax
import jax.numpy as jnp
from jax import lax
from jax.experimental import pallas as pl
from jax.experimental.pallas import tpu as pltpu

_BR = 128

_DN = (((1,), (0,)), ((), ()))


def _mm(a, b):
    return lax.dot_general(a, b, _DN, preferred_element_type=jnp.float32)


def _body(dxT_ref, inpT_ref, procT_ref, demoT_ref,
          wdx_ref, bdx_ref, winp_ref, binp_ref, wproc_ref, bproc_ref,
          wdemo_ref, bdemo_ref, wfi_ref, wfp_ref, wfd_ref, bf_ref,
          dx0T_ref, outT_ref):
    T = inpT_ref.shape[0]

    dx0T_ref[...] = jnp.tanh(_mm(wdx_ref[...], dxT_ref[...]) + bdx_ref[...])

    de = jnp.tanh(_mm(wdemo_ref[...], demoT_ref[...]) + bdemo_ref[...])
    demo_part = _mm(wfd_ref[...], de) + bf_ref[...]

    winp = winp_ref[...]
    wproc = wproc_ref[...]
    binp = binp_ref[...]
    bproc = bproc_ref[...]
    wfi = wfi_ref[...]
    wfp = wfp_ref[...]
    for t in range(T):
        ie = jnp.tanh(_mm(winp, inpT_ref[t]) + binp)
        pe = jnp.tanh(_mm(wproc, procT_ref[t]) + bproc)
        outT_ref[:, t, :] = jnp.tanh(_mm(wfi, ie) + _mm(wfp, pe) + demo_part)


def kernel(dx_history, inp, proc, demo, W_dx, b_dx, W_inp, b_inp,
           W_proc, b_proc, W_demo, b_demo, W_f, b_f):
    B, DXV = dx_history.shape
    _, T, IV = inp.shape
    PV = proc.shape[2]
    DD = demo.shape[1]
    DXE = W_dx.shape[0]
    IE = W_inp.shape[0]
    PE = W_proc.shape[0]
    DE = W_demo.shape[0]
    FE = W_f.shape[0]

    dxT = dx_history.T
    inpT = jnp.transpose(inp, (1, 2, 0))
    procT = jnp.transpose(proc, (1, 2, 0))
    demoT = demo.T

    wfi = W_f[:, :IE]
    wfp = W_f[:, IE:IE + PE]
    wfd = W_f[:, IE + PE:]
    bdx2 = b_dx.reshape(DXE, 1)
    binp2 = b_inp.reshape(IE, 1)
    bproc2 = b_proc.reshape(PE, 1)
    bdemo2 = b_demo.reshape(DE, 1)
    bf2 = b_f.reshape(FE, 1)

    grid = (B // _BR,)
    full = lambda shape: pl.BlockSpec(shape, lambda i: (0,) * len(shape))

    dx0T, outT = pl.pallas_call(
        _body,
        grid=grid,
        in_specs=[
            pl.BlockSpec((DXV, _BR), lambda i: (0, i)),
            pl.BlockSpec((T, IV, _BR), lambda i: (0, 0, i)),
            pl.BlockSpec((T, PV, _BR), lambda i: (0, 0, i)),
            pl.BlockSpec((DD, _BR), lambda i: (0, i)),
            full((DXE, DXV)), full((DXE, 1)),
            full((IE, IV)), full((IE, 1)),
            full((PE, PV)), full((PE, 1)),
            full((DE, DD)), full((DE, 1)),
            full((FE, IE)), full((FE, PE)), full((FE, DE)), full((FE, 1)),
        ],
        out_specs=[
            pl.BlockSpec((DXE, _BR), lambda i: (0, i)),
            pl.BlockSpec((FE, T, _BR), lambda i: (0, 0, i)),
        ],
        out_shape=[
            jax.ShapeDtypeStruct((DXE, B), jnp.float32),
            jax.ShapeDtypeStruct((FE, T, B), jnp.float32),
        ],
        compiler_params=pltpu.CompilerParams(
            dimension_semantics=("arbitrary",),
        ),
    )(dxT, inpT, procT, demoT,
      W_dx, bdx2, W_inp, binp2, W_proc, bproc2,
      W_demo, bdemo2, wfi, wfp, wfd, bf2)
    return dx0T.T, jnp.transpose(outT, (2, 1, 0))
